# baseline (speedup 1.0000x reference)
import functools
import math

import jax
import jax.numpy as jnp
from jax import lax
from jax.experimental import pallas as pl
from jax.experimental.pallas import tpu as pltpu
from jax.experimental.pallas import tpu_sc as plsc

VOCAB_SIZE = 1_000_000
D_MODEL = 64
SCALE = math.sqrt(D_MODEL)

NUM_CORES = 2
NUM_SUBCORES = 16
LANES = 16
NUM_WORKERS = NUM_CORES * NUM_SUBCORES

CHUNK = 128


def _make_lookup(batch_flat: int):
  assert batch_flat % (NUM_WORKERS * CHUNK) == 0
  per_worker = batch_flat // NUM_WORKERS
  n_chunks = per_worker // CHUNK

  mesh = plsc.VectorSubcoreMesh(
      core_axis_name="c", subcore_axis_name="s",
      num_cores=NUM_CORES, num_subcores=NUM_SUBCORES)

  @functools.partial(
      pl.kernel,
      out_type=jax.ShapeDtypeStruct((batch_flat, D_MODEL), jnp.float32),
      mesh=mesh,
      scratch_types=[
          pltpu.VMEM((CHUNK,), jnp.int32),
          pltpu.VMEM((CHUNK, D_MODEL), jnp.float32),
          pltpu.SemaphoreType.DMA,
      ],
      compiler_params=pltpu.CompilerParams(use_tc_tiling_on_sc=False),
  )
  def lookup(table_hbm, idx_hbm, out_hbm, idx_v, rows_v, sem):
    wid = lax.axis_index("s") * NUM_CORES + lax.axis_index("c")
    base = wid * per_worker

    @pl.loop(0, n_chunks)
    def _chunk(g):
      row0 = base + g * CHUNK
      pltpu.sync_copy(idx_hbm.at[pl.ds(row0, CHUNK)], idx_v)
      pltpu.async_copy(table_hbm.at[idx_v], rows_v, sem).wait()

      @pl.loop(0, CHUNK)
      def _scale(r):
        for j in range(D_MODEL // LANES):
          sl = (r, pl.ds(j * LANES, LANES))
          rows_v[sl] = rows_v[sl] * SCALE

      pltpu.sync_copy(rows_v, out_hbm.at[pl.ds(row0, CHUNK)])

  return lookup


def kernel(x, table):
  batch_shape = x.shape
  x_flat = x.reshape(-1).astype(jnp.int32)
  out = _make_lookup(x_flat.shape[0])(table, x_flat)
  return out.reshape(*batch_shape, D_MODEL)


# speedup vs baseline: 1.2252x; 1.2252x over previous
import functools
import math

import jax
import jax.numpy as jnp
from jax import lax
from jax.experimental import pallas as pl
from jax.experimental.pallas import tpu as pltpu
from jax.experimental.pallas import tpu_sc as plsc

VOCAB_SIZE = 1_000_000
D_MODEL = 64
SCALE = math.sqrt(D_MODEL)

NUM_CORES = 2
NUM_SUBCORES = 16
LANES = 16
NUM_WORKERS = NUM_CORES * NUM_SUBCORES

CHUNK = 128
NBUF = 8


def _make_lookup(batch_flat: int):
  assert batch_flat % (NUM_WORKERS * CHUNK * NBUF) == 0
  per_worker = batch_flat // NUM_WORKERS
  n_chunks = per_worker // CHUNK

  mesh = plsc.VectorSubcoreMesh(
      core_axis_name="c", subcore_axis_name="s",
      num_cores=NUM_CORES, num_subcores=NUM_SUBCORES)

  @functools.partial(
      pl.kernel,
      out_type=jax.ShapeDtypeStruct((batch_flat, D_MODEL), jnp.float32),
      mesh=mesh,
      scratch_types=[
          pltpu.VMEM((n_chunks, CHUNK), jnp.int32),
          pltpu.VMEM((NBUF, CHUNK, D_MODEL), jnp.float32),
          pltpu.SemaphoreType.DMA((NBUF,)),
          pltpu.SemaphoreType.DMA((NBUF,)),
      ],
      compiler_params=pltpu.CompilerParams(use_tc_tiling_on_sc=False),
  )
  def lookup(table_hbm, idx_hbm, out_hbm, idx_all, rows, gsem, ssem):
    wid = lax.axis_index("s") * NUM_CORES + lax.axis_index("c")
    base = wid * per_worker
    pltpu.sync_copy(idx_hbm.at[pl.ds(wid * n_chunks, n_chunks)], idx_all)

    @pl.loop(0, n_chunks, step=NBUF)
    def _group(g0):
      gathers = [
          pltpu.async_copy(
              table_hbm.at[idx_all.at[g0 + b]], rows.at[b], gsem.at[b])
          for b in range(NBUF)
      ]
      stores = []
      for b in range(NBUF):
        gathers[b].wait()
        row_buf = rows.at[b]

        @pl.loop(0, CHUNK)
        def _scale(r):
          for j in range(D_MODEL // LANES):
            sl = (r, pl.ds(j * LANES, LANES))
            row_buf[sl] = row_buf[sl] * SCALE

        stores.append(
            pltpu.async_copy(
                row_buf, out_hbm.at[pl.ds(base + (g0 + b) * CHUNK, CHUNK)],
                ssem.at[b]))
      for st in stores:
        st.wait()

  return lookup


def kernel(x, table):
  batch_shape = x.shape
  x_flat = x.reshape(-1).astype(jnp.int32)
  idx2d = x_flat.reshape(-1, CHUNK)
  out = _make_lookup(x_flat.shape[0])(table, idx2d)
  return out.reshape(*batch_shape, D_MODEL)
